# baseline (device time: 35582 ns/iter reference)
import jax
import jax.numpy as jnp
from jax import lax
from jax.experimental import pallas as pl
from jax.experimental.pallas import tpu as pltpu


def kernel(Q, K, V):
    b, q_len, h, d = Q.shape
    scale = d ** -0.5

    def body(q_ref, k_ref, v_ref, out_ref,
             o_send, o_recv, l_send, l_recv, send_sems, recv_sems):
        my_x = lax.axis_index("x")
        my_y = lax.axis_index("y")
        my_z = lax.axis_index("z")
        partner = (my_x, 1 - my_y, my_z)

        q = q_ref[:, 0, :, :].astype(jnp.float32)
        k = k_ref[...].astype(jnp.float32)
        s = jnp.sum(q[:, None, :, :] * k, axis=-1) * scale
        p = jnp.exp(s)
        l_send[...] = jnp.sum(p, axis=1)
        v = v_ref[...].astype(jnp.float32)
        o_send[...] = jnp.sum(p[:, :, :, None] * v, axis=1)

        barrier_sem = pltpu.get_barrier_semaphore()
        pl.semaphore_signal(barrier_sem, inc=1, device_id=partner,
                            device_id_type=pl.DeviceIdType.MESH)
        pl.semaphore_wait(barrier_sem, 1)

        o_rdma = pltpu.make_async_remote_copy(
            src_ref=o_send, dst_ref=o_recv,
            send_sem=send_sems.at[0], recv_sem=recv_sems.at[0],
            device_id=partner, device_id_type=pl.DeviceIdType.MESH)
        l_rdma = pltpu.make_async_remote_copy(
            src_ref=l_send, dst_ref=l_recv,
            send_sem=send_sems.at[1], recv_sem=recv_sems.at[1],
            device_id=partner, device_id_type=pl.DeviceIdType.MESH)
        o_rdma.start()
        l_rdma.start()
        o_rdma.wait()
        l_rdma.wait()

        l_tot = l_send[...] + l_recv[...]
        o_tot = o_send[...] + o_recv[...]
        out_ref[:, 0, :, :] = o_tot / l_tot[:, :, None]

    return pl.pallas_call(
        body,
        out_shape=jax.ShapeDtypeStruct((b, q_len, h, d), jnp.float32),
        in_specs=[pl.BlockSpec(memory_space=pltpu.VMEM)] * 3,
        out_specs=pl.BlockSpec(memory_space=pltpu.VMEM),
        scratch_shapes=[
            pltpu.VMEM((b, h, d), jnp.float32),
            pltpu.VMEM((b, h, d), jnp.float32),
            pltpu.VMEM((b, h), jnp.float32),
            pltpu.VMEM((b, h), jnp.float32),
            pltpu.SemaphoreType.DMA((2,)),
            pltpu.SemaphoreType.DMA((2,)),
        ],
        compiler_params=pltpu.CompilerParams(collective_id=0),
    )(Q, K, V)


# device time: 19310 ns/iter; 1.8427x vs baseline; 1.8427x over previous
import jax
import jax.numpy as jnp
from jax import lax
from jax.experimental import pallas as pl
from jax.experimental.pallas import tpu as pltpu


def kernel(Q, K, V):
    b, q_len, h, d = Q.shape
    kk = K.shape[1]
    hd = h * d
    scale = d ** -0.5
    Q2 = Q.reshape(b, hd)
    K2 = K.reshape(b, kk, hd)
    V2 = V.reshape(b, kk, hd)

    def body(q_ref, k_ref, v_ref, out_ref,
             o_send, o_recv, l_send, l_recv, send_sems, recv_sems):
        my_x = lax.axis_index("x")
        my_y = lax.axis_index("y")
        my_z = lax.axis_index("z")
        partner = (my_x, 1 - my_y, my_z)

        col_h = lax.broadcasted_iota(jnp.int32, (h, hd), 1) // d
        row_h = lax.broadcasted_iota(jnp.int32, (h, hd), 0)
        maskt = (row_h == col_h).astype(jnp.float32)

        qm = q_ref[...][:, None, :] * maskt[None]
        s = lax.dot_general(
            qm, k_ref[...],
            (((2,), (2,)), ((0,), (0,))),
            preferred_element_type=jnp.float32,
        ) * scale
        p = jnp.exp(s)
        l_send[...] = jnp.sum(p, axis=2)
        r = lax.dot_general(
            p, v_ref[...],
            (((2,), (1,)), ((0,), (0,))),
            preferred_element_type=jnp.float32,
        )
        o_send[...] = jnp.sum(r * maskt[None], axis=1)

        barrier_sem = pltpu.get_barrier_semaphore()
        pl.semaphore_signal(barrier_sem, inc=1, device_id=partner,
                            device_id_type=pl.DeviceIdType.MESH)
        pl.semaphore_wait(barrier_sem, 1)

        o_rdma = pltpu.make_async_remote_copy(
            src_ref=o_send, dst_ref=o_recv,
            send_sem=send_sems.at[0], recv_sem=recv_sems.at[0],
            device_id=partner, device_id_type=pl.DeviceIdType.MESH)
        l_rdma = pltpu.make_async_remote_copy(
            src_ref=l_send, dst_ref=l_recv,
            send_sem=send_sems.at[1], recv_sem=recv_sems.at[1],
            device_id=partner, device_id_type=pl.DeviceIdType.MESH)
        o_rdma.start()
        l_rdma.start()
        o_rdma.wait()
        l_rdma.wait()

        l_tot = l_send[...] + l_recv[...]
        o_tot = o_send[...] + o_recv[...]
        l_exp = lax.dot_general(
            l_tot, maskt,
            (((1,), (0,)), ((), ())),
            preferred_element_type=jnp.float32,
        )
        out_ref[...] = o_tot / l_exp

    out = pl.pallas_call(
        body,
        out_shape=jax.ShapeDtypeStruct((b, hd), jnp.float32),
        in_specs=[pl.BlockSpec(memory_space=pltpu.VMEM)] * 3,
        out_specs=pl.BlockSpec(memory_space=pltpu.VMEM),
        scratch_shapes=[
            pltpu.VMEM((b, hd), jnp.float32),
            pltpu.VMEM((b, hd), jnp.float32),
            pltpu.VMEM((b, h), jnp.float32),
            pltpu.VMEM((b, h), jnp.float32),
            pltpu.SemaphoreType.DMA((2,)),
            pltpu.SemaphoreType.DMA((2,)),
        ],
        compiler_params=pltpu.CompilerParams(collective_id=0),
    )(Q2, K2, V2)
    return out.reshape(b, q_len, h, d)
